# Initial kernel scaffold; baseline (speedup 1.0000x reference)
import jax
import jax.numpy as jnp
from jax.experimental import pallas as pl


def kernel(node_feats, edge_index, W1, al1, ar1, g0, b0, g1, b1, W2, al2, ar2, g2, b2, W3, al3, ar3, Wc):
    raise NotImplementedError("write your pallas kernel here")


# speedup vs baseline: 67.6631x; 67.6631x over previous
import functools

import numpy as np
import jax
import jax.numpy as jnp
from jax import lax
from jax.experimental import pallas as pl
from jax.experimental.pallas import tpu as pltpu, tpu_sc as plsc

N = 10000
E = 320000
HP = 16
C = 80
NW = 32
NSUB = 16
ECT = E // C
CPT = ECT // NW
NP = 10240
RPT = NP // NSUB


def _expander(h, fo, width):
    z = np.zeros((HP, width), np.float32)
    for i in range(h):
        z[i, i * fo:(i + 1) * fo] = 1.0
    return z


def _tc1_body(x_ref, g_ref, b_ref, w_ref, al_ref, ar_ref, s_ref,
              ro_ref, feat_ref, el_ref, er_ref):
    x = x_ref[...]
    r = jnp.maximum(x, 0.01 * x)
    mu = jnp.mean(r, axis=0, keepdims=True)
    var = jnp.mean((r - mu) ** 2, axis=0, keepdims=True)
    rn = g_ref[...] * (r - mu) / jnp.sqrt(var + 1e-5) + b_ref[...]
    ro_ref[...] = rn
    feat = jnp.dot(rn, w_ref[...], preferred_element_type=jnp.float32)
    feat_ref[...] = feat
    el_ref[...] = jnp.dot(feat * al_ref[...], s_ref[...],
                          preferred_element_type=jnp.float32)
    er_ref[...] = jnp.dot(feat * ar_ref[...], s_ref[...],
                          preferred_element_type=jnp.float32)


def _tc_mid_body(ap_ref, dp_ref, em_ref, g_ref, b_ref, w_ref, al_ref, ar_ref,
                 s_ref, feat_ref, el_ref, er_ref):
    acc = ap_ref[0, :N, :] + ap_ref[1, :N, :]
    den = dp_ref[0, :N, :] + dp_ref[1, :N, :]
    dnx = jnp.dot(den, em_ref[...], preferred_element_type=jnp.float32)
    h = acc / (dnx + 1e-9)
    h = jnp.maximum(h, 0.01 * h)
    mu = jnp.mean(h, axis=0, keepdims=True)
    var = jnp.mean((h - mu) ** 2, axis=0, keepdims=True)
    hn = g_ref[...] * (h - mu) / jnp.sqrt(var + 1e-5) + b_ref[...]
    feat = jnp.dot(hn, w_ref[...], preferred_element_type=jnp.float32)
    feat_ref[...] = feat
    el_ref[...] = jnp.dot(feat * al_ref[...], s_ref[...],
                          preferred_element_type=jnp.float32)
    er_ref[...] = jnp.dot(feat * ar_ref[...], s_ref[...],
                          preferred_element_type=jnp.float32)


def _tc_fin_body(ap_ref, dp_ref, em_ref, wc_ref, out_ref):
    acc = ap_ref[0, :N, :] + ap_ref[1, :N, :]
    den = dp_ref[0, :N, :] + dp_ref[1, :N, :]
    dnx = jnp.dot(den, em_ref[...], preferred_element_type=jnp.float32)
    o = acc / (dnx + 1e-9)
    hm = (o[:, 0:32] + o[:, 32:64]) * 0.5
    pooled = jnp.mean(hm, axis=0, keepdims=True)
    out_ref[...] = jnp.dot(pooled, wc_ref[...],
                           preferred_element_type=jnp.float32)


def _tc1(x, g, b, w, al, ar, sel):
    fo = w.shape[1]
    return pl.pallas_call(
        _tc1_body,
        out_shape=[
            jax.ShapeDtypeStruct((N, 32), jnp.float32),
            jax.ShapeDtypeStruct((N, fo), jnp.float32),
            jax.ShapeDtypeStruct((N, HP), jnp.float32),
            jax.ShapeDtypeStruct((N, HP), jnp.float32),
        ],
    )(x, g, b, w, al, ar, sel)


def _tc_mid(ap, dp, em, g, b, w, al, ar, sel):
    fo = w.shape[1]
    return pl.pallas_call(
        _tc_mid_body,
        out_shape=[
            jax.ShapeDtypeStruct((N, fo), jnp.float32),
            jax.ShapeDtypeStruct((N, HP), jnp.float32),
            jax.ShapeDtypeStruct((N, HP), jnp.float32),
        ],
    )(ap, dp, em, g, b, w, al, ar, sel)


def _tc_fin(ap, dp, em, wc):
    return pl.pallas_call(
        _tc_fin_body,
        out_shape=jax.ShapeDtypeStruct((1, 15), jnp.float32),
    )(ap, dp, em, wc)


def _make_sc_pass_a():
    mesh = plsc.VectorSubcoreMesh(core_axis_name="c", subcore_axis_name="s")

    @functools.partial(
        pl.kernel,
        out_type=[
            jax.ShapeDtypeStruct((ECT, C, HP), jnp.float32),
            jax.ShapeDtypeStruct((2, NP, HP), jnp.float32),
        ],
        mesh=mesh,
        compiler_params=pltpu.CompilerParams(use_tc_tiling_on_sc=False),
        scratch_types=[
            pltpu.VMEM((CPT, C), jnp.int32),
            pltpu.VMEM((CPT, C), jnp.int32),
            pltpu.VMEM((C, HP), jnp.float32),
            pltpu.VMEM((C, HP), jnp.float32),
            pltpu.VMEM((C, HP), jnp.float32),
            pltpu.VMEM_SHARED((NP, HP), jnp.float32),
            pltpu.SemaphoreType.DMA,
        ],
    )
    def kern(el_hbm, er_hbm, src_hbm, dst_hbm, s_out, den_out,
             src_v, dst_v, el_v, er_v, s_v, den_sh, sem):
        cid = lax.axis_index("c")
        sid = lax.axis_index("s")
        wid = cid * NSUB + sid
        pltpu.sync_copy(src_hbm.at[wid], src_v)
        pltpu.sync_copy(dst_hbm.at[wid], dst_v)

        def zero_body(k, carry):
            s_v[k] = jnp.zeros((HP,), jnp.float32)
            return carry
        lax.fori_loop(0, C, zero_body, None)
        for i in range(RPT // C):
            pltpu.sync_copy(s_v, den_sh.at[pl.ds(sid * RPT + i * C, C)])
        plsc.subcore_barrier()

        def chunk(b, carry):
            pltpu.async_copy(el_hbm.at[src_v.at[b]], el_v, sem).wait()
            pltpu.async_copy(er_hbm.at[dst_v.at[b]], er_v, sem).wait()

            def edge(k, c2):
                z = el_v[k] + er_v[k]
                s_v[k] = jnp.exp(jnp.maximum(z, 0.2 * z))
                return c2
            lax.fori_loop(0, C, edge, None)
            pltpu.sync_copy(s_v, s_out.at[wid * CPT + b])
            pltpu.sync_copy(s_v, den_sh.at[dst_v.at[b]], add=True)
            return carry
        lax.fori_loop(0, CPT, chunk, None)
        plsc.subcore_barrier()
        for i in range(RPT // C):
            r0 = sid * RPT + i * C
            pltpu.sync_copy(den_sh.at[pl.ds(r0, C)],
                            den_out.at[cid, pl.ds(r0, C)])

    return kern


def _make_sc_pass_b(f, h):
    nv = f // 16
    mesh = plsc.VectorSubcoreMesh(core_axis_name="c", subcore_axis_name="s")

    @functools.partial(
        pl.kernel,
        out_type=jax.ShapeDtypeStruct((2, NP, f), jnp.float32),
        mesh=mesh,
        compiler_params=pltpu.CompilerParams(use_tc_tiling_on_sc=False),
        scratch_types=[
            pltpu.VMEM((CPT, C), jnp.int32),
            pltpu.VMEM((CPT, C), jnp.int32),
            pltpu.VMEM((2, C, f), jnp.float32),
            pltpu.VMEM((C, HP), jnp.float32),
            pltpu.VMEM_SHARED((NP, f), jnp.float32),
            pltpu.SemaphoreType.DMA,
            pltpu.SemaphoreType.DMA,
        ],
    )
    def kern(feat_hbm, s_hbm, src_hbm, dst_hbm, acc_out,
             src_v, dst_v, rows_v, s_v, acc_sh, sem0, sem1):
        cid = lax.axis_index("c")
        sid = lax.axis_index("s")
        wid = cid * NSUB + sid
        sems = (sem0, sem1)
        pltpu.sync_copy(src_hbm.at[wid], src_v)
        pltpu.sync_copy(dst_hbm.at[wid], dst_v)

        def zero_body(k, carry):
            for j in range(nv):
                rows_v[0, k, pl.ds(j * 16, 16)] = jnp.zeros((16,), jnp.float32)
            return carry
        lax.fori_loop(0, C, zero_body, None)
        for i in range(RPT // C):
            pltpu.sync_copy(rows_v.at[0], acc_sh.at[pl.ds(sid * RPT + i * C, C)])
        plsc.subcore_barrier()

        pltpu.async_copy(feat_hbm.at[src_v.at[0]], rows_v.at[0], sem0)

        def process(b, buf, nxt_buf, do_prefetch):
            pltpu.make_async_copy(feat_hbm.at[src_v.at[b]],
                                  rows_v.at[buf], sems[buf]).wait()
            if do_prefetch:
                pltpu.async_copy(feat_hbm.at[src_v.at[b + 1]],
                                 rows_v.at[nxt_buf], sems[nxt_buf])
            pltpu.sync_copy(s_hbm.at[wid * CPT + b], s_v)

            def edge(k, c2):
                srow = s_v[k]
                for hh in range(h):
                    w = jnp.broadcast_to(srow[hh], (16,))
                    for j in range(2):
                        col = hh * 32 + j * 16
                        rows_v[buf, k, pl.ds(col, 16)] = (
                            rows_v[buf, k, pl.ds(col, 16)] * w)
                return c2
            lax.fori_loop(0, C, edge, None, unroll=2)
            pltpu.sync_copy(rows_v.at[buf], acc_sh.at[dst_v.at[b]], add=True)

        def pair(p, carry):
            b0 = p * 2
            process(b0, 0, 1, True)
            process(b0 + 1, 1, 0, True)
            return carry
        lax.fori_loop(0, (CPT - 1) // 2, pair, None)
        process(CPT - 1, (CPT - 1) % 2, 0, False)
        plsc.subcore_barrier()
        for i in range(RPT // C):
            r0 = sid * RPT + i * C
            pltpu.sync_copy(acc_sh.at[pl.ds(r0, C)],
                            acc_out.at[cid, pl.ds(r0, C)])

    return kern


def _make_sc_edge(f, h):
    nv = f // 16
    mesh = plsc.VectorSubcoreMesh(core_axis_name="c", subcore_axis_name="s")

    @functools.partial(
        pl.kernel,
        out_type=[
            jax.ShapeDtypeStruct((2, NP, HP), jnp.float32),
            jax.ShapeDtypeStruct((2, NP, f), jnp.float32),
        ],
        mesh=mesh,
        compiler_params=pltpu.CompilerParams(use_tc_tiling_on_sc=False),
        scratch_types=[
            pltpu.VMEM((CPT, C), jnp.int32),
            pltpu.VMEM((CPT, C), jnp.int32),
            pltpu.VMEM((2, C, f), jnp.float32),
            pltpu.VMEM((2, C, HP), jnp.float32),
            pltpu.VMEM((2, C, HP), jnp.float32),
            pltpu.VMEM((C, HP), jnp.float32),
            pltpu.VMEM_SHARED((NP, HP), jnp.float32),
            pltpu.VMEM_SHARED((NP, f), jnp.float32),
            pltpu.SemaphoreType.DMA,
            pltpu.SemaphoreType.DMA,
        ],
    )
    def kern(feat_hbm, el_hbm, er_hbm, src_hbm, dst_hbm, den_out, acc_out,
             src_v, dst_v, rows_v, el_v, er_v, s_v, den_sh, acc_sh,
             sem0, sem1):
        cid = lax.axis_index("c")
        sid = lax.axis_index("s")
        wid = cid * NSUB + sid
        sems = (sem0, sem1)
        pltpu.sync_copy(src_hbm.at[wid], src_v)
        pltpu.sync_copy(dst_hbm.at[wid], dst_v)

        def zero_body(k, carry):
            s_v[k] = jnp.zeros((HP,), jnp.float32)
            for j in range(nv):
                rows_v[0, k, pl.ds(j * 16, 16)] = jnp.zeros((16,), jnp.float32)
            return carry
        lax.fori_loop(0, C, zero_body, None)
        for i in range(RPT // C):
            r0 = sid * RPT + i * C
            pltpu.sync_copy(s_v, den_sh.at[pl.ds(r0, C)])
            pltpu.sync_copy(rows_v.at[0], acc_sh.at[pl.ds(r0, C)])
        plsc.subcore_barrier()

        def prefetch(b, buf):
            pltpu.async_copy(feat_hbm.at[src_v.at[b]], rows_v.at[buf],
                             sems[buf])
            pltpu.async_copy(el_hbm.at[src_v.at[b]], el_v.at[buf], sems[buf])
            pltpu.async_copy(er_hbm.at[dst_v.at[b]], er_v.at[buf], sems[buf])

        def wait_buf(b, buf):
            pltpu.make_async_copy(feat_hbm.at[src_v.at[b]], rows_v.at[buf],
                                  sems[buf]).wait()
            pltpu.make_async_copy(el_hbm.at[src_v.at[b]], el_v.at[buf],
                                  sems[buf]).wait()
            pltpu.make_async_copy(er_hbm.at[dst_v.at[b]], er_v.at[buf],
                                  sems[buf]).wait()

        def process(b, buf, do_prefetch):
            wait_buf(b, buf)
            if do_prefetch:
                prefetch(b + 1, 1 - buf)

            def edge(k, c2):
                z = el_v[buf, k] + er_v[buf, k]
                srow = jnp.exp(jnp.maximum(z, 0.2 * z))
                s_v[k] = srow
                for hh in range(h):
                    w = jnp.broadcast_to(srow[hh], (16,))
                    for j in range(2):
                        col = hh * 32 + j * 16
                        rows_v[buf, k, pl.ds(col, 16)] = (
                            rows_v[buf, k, pl.ds(col, 16)] * w)
                return c2
            lax.fori_loop(0, C, edge, None, unroll=2)
            pltpu.sync_copy(s_v, den_sh.at[dst_v.at[b]], add=True)
            pltpu.sync_copy(rows_v.at[buf], acc_sh.at[dst_v.at[b]], add=True)

        prefetch(0, 0)

        def pair(p, carry):
            b0 = p * 2
            process(b0, 0, True)
            process(b0 + 1, 1, True)
            return carry
        lax.fori_loop(0, (CPT - 1) // 2, pair, None)
        process(CPT - 1, (CPT - 1) % 2, False)

        plsc.subcore_barrier()
        for i in range(RPT // C):
            r0 = sid * RPT + i * C
            pltpu.sync_copy(den_sh.at[pl.ds(r0, C)],
                            den_out.at[cid, pl.ds(r0, C)])
            pltpu.sync_copy(acc_sh.at[pl.ds(r0, C)],
                            acc_out.at[cid, pl.ds(r0, C)])

    return kern


_SC_A = _make_sc_pass_a()
_SC_B128 = _make_sc_pass_b(128, 4)
_SC_EDGE = {f: _make_sc_edge(f, f // 32) for f in (96, 64)}


def kernel(node_feats, edge_index, W1, al1, ar1, g0, b0, g1, b1,
           W2, al2, ar2, g2, b2, W3, al3, ar3, Wc):
    f32 = jnp.float32
    src3d = edge_index[0].reshape(NW, CPT, C)
    dst3d = edge_index[1].reshape(NW, CPT, C)

    sel1 = jnp.asarray(_expander(4, 32, 128).T)
    sel2 = jnp.asarray(_expander(3, 32, 96).T)
    sel3 = jnp.asarray(_expander(2, 32, 64).T)
    em1 = jnp.asarray(_expander(4, 32, 128))
    em2 = jnp.asarray(_expander(3, 32, 96))
    em3 = jnp.asarray(_expander(2, 32, 64))

    ro, feat1, el1, er1 = _tc1(
        node_feats, g0.reshape(1, -1), b0.reshape(1, -1), W1,
        al1.reshape(1, -1).astype(f32), ar1.reshape(1, -1).astype(f32), sel1)
    s1, den1 = _SC_A(el1, er1, src3d, dst3d)
    acc1 = _SC_B128(feat1, s1, src3d, dst3d)

    feat2, el2, er2 = _tc_mid(
        acc1, den1, em1, g1.reshape(1, -1), b1.reshape(1, -1), W2,
        al2.reshape(1, -1).astype(f32), ar2.reshape(1, -1).astype(f32), sel2)
    den2, acc2 = _SC_EDGE[96](feat2, el2, er2, src3d, dst3d)

    feat3, el3, er3 = _tc_mid(
        acc2, den2, em2, g2.reshape(1, -1), b2.reshape(1, -1), W3,
        al3.reshape(1, -1).astype(f32), ar3.reshape(1, -1).astype(f32), sel3)
    den3, acc3 = _SC_EDGE[64](feat3, el3, er3, src3d, dst3d)

    logits = _tc_fin(acc3, den3, em3, Wc)
    return (logits, ro)


# speedup vs baseline: 72.2375x; 1.0676x over previous
import functools

import numpy as np
import jax
import jax.numpy as jnp
from jax import lax
from jax.experimental import pallas as pl
from jax.experimental.pallas import tpu as pltpu, tpu_sc as plsc

N = 10000
E = 320000
HP = 16
C = 80
NW = 32
NSUB = 16
ECT = E // C
CPT = ECT // NW
NP = 10240
RPT = NP // NSUB


def _expander(h, fo, width):
    z = np.zeros((HP, width), np.float32)
    for i in range(h):
        z[i, i * fo:(i + 1) * fo] = 1.0
    return z


def _tc1_body(x_ref, g_ref, b_ref, w_ref, al_ref, ar_ref, s_ref,
              ro_ref, feat_ref, el_ref, er_ref):
    x = x_ref[...]
    r = jnp.maximum(x, 0.01 * x)
    mu = jnp.mean(r, axis=0, keepdims=True)
    var = jnp.mean((r - mu) ** 2, axis=0, keepdims=True)
    rn = g_ref[...] * (r - mu) / jnp.sqrt(var + 1e-5) + b_ref[...]
    ro_ref[...] = rn
    feat = jnp.dot(rn, w_ref[...], preferred_element_type=jnp.float32)
    feat_ref[...] = feat
    el_ref[...] = jnp.dot(feat * al_ref[...], s_ref[...],
                          preferred_element_type=jnp.float32)
    er_ref[...] = jnp.dot(feat * ar_ref[...], s_ref[...],
                          preferred_element_type=jnp.float32)


def _tc_mid_body(ap_ref, dp_ref, em_ref, g_ref, b_ref, w_ref, al_ref, ar_ref,
                 s_ref, feat_ref, el_ref, er_ref):
    acc = ap_ref[0, :N, :] + ap_ref[1, :N, :]
    den = dp_ref[0, :N, :] + dp_ref[1, :N, :]
    dnx = jnp.dot(den, em_ref[...], preferred_element_type=jnp.float32)
    h = acc / (dnx + 1e-9)
    h = jnp.maximum(h, 0.01 * h)
    mu = jnp.mean(h, axis=0, keepdims=True)
    var = jnp.mean((h - mu) ** 2, axis=0, keepdims=True)
    hn = g_ref[...] * (h - mu) / jnp.sqrt(var + 1e-5) + b_ref[...]
    feat = jnp.dot(hn, w_ref[...], preferred_element_type=jnp.float32)
    feat_ref[...] = feat
    el_ref[...] = jnp.dot(feat * al_ref[...], s_ref[...],
                          preferred_element_type=jnp.float32)
    er_ref[...] = jnp.dot(feat * ar_ref[...], s_ref[...],
                          preferred_element_type=jnp.float32)


def _tc_fin_body(ap_ref, dp_ref, em_ref, wc_ref, out_ref):
    acc = ap_ref[0, :N, :] + ap_ref[1, :N, :]
    den = dp_ref[0, :N, :] + dp_ref[1, :N, :]
    dnx = jnp.dot(den, em_ref[...], preferred_element_type=jnp.float32)
    o = acc / (dnx + 1e-9)
    hm = (o[:, 0:32] + o[:, 32:64]) * 0.5
    pooled = jnp.mean(hm, axis=0, keepdims=True)
    out_ref[...] = jnp.dot(pooled, wc_ref[...],
                           preferred_element_type=jnp.float32)


def _tc1(x, g, b, w, al, ar, sel):
    fo = w.shape[1]
    return pl.pallas_call(
        _tc1_body,
        out_shape=[
            jax.ShapeDtypeStruct((N, 32), jnp.float32),
            jax.ShapeDtypeStruct((N, fo), jnp.float32),
            jax.ShapeDtypeStruct((N, HP), jnp.float32),
            jax.ShapeDtypeStruct((N, HP), jnp.float32),
        ],
    )(x, g, b, w, al, ar, sel)


def _tc_mid(ap, dp, em, g, b, w, al, ar, sel):
    fo = w.shape[1]
    return pl.pallas_call(
        _tc_mid_body,
        out_shape=[
            jax.ShapeDtypeStruct((N, fo), jnp.float32),
            jax.ShapeDtypeStruct((N, HP), jnp.float32),
            jax.ShapeDtypeStruct((N, HP), jnp.float32),
        ],
    )(ap, dp, em, g, b, w, al, ar, sel)


def _tc_fin(ap, dp, em, wc):
    return pl.pallas_call(
        _tc_fin_body,
        out_shape=jax.ShapeDtypeStruct((1, 15), jnp.float32),
    )(ap, dp, em, wc)


def _make_sc_pass_a():
    mesh = plsc.VectorSubcoreMesh(core_axis_name="c", subcore_axis_name="s")

    @functools.partial(
        pl.kernel,
        out_type=[
            jax.ShapeDtypeStruct((ECT, C, HP), jnp.float32),
            jax.ShapeDtypeStruct((2, NP, HP), jnp.float32),
        ],
        mesh=mesh,
        compiler_params=pltpu.CompilerParams(use_tc_tiling_on_sc=False),
        scratch_types=[
            pltpu.VMEM((CPT, C), jnp.int32),
            pltpu.VMEM((CPT, C), jnp.int32),
            pltpu.VMEM((2, C, HP), jnp.float32),
            pltpu.VMEM((2, C, HP), jnp.float32),
            pltpu.VMEM((C, HP), jnp.float32),
            pltpu.VMEM_SHARED((NP, HP), jnp.float32),
            pltpu.SemaphoreType.DMA,
            pltpu.SemaphoreType.DMA,
        ],
    )
    def kern(el_hbm, er_hbm, src_hbm, dst_hbm, s_out, den_out,
             src_v, dst_v, el_v, er_v, s_v, den_sh, sem, sem2):
        cid = lax.axis_index("c")
        sid = lax.axis_index("s")
        wid = cid * NSUB + sid
        pltpu.sync_copy(src_hbm.at[wid], src_v)
        pltpu.sync_copy(dst_hbm.at[wid], dst_v)

        def zero_body(k, carry):
            s_v[k] = jnp.zeros((HP,), jnp.float32)
            return carry
        lax.fori_loop(0, C, zero_body, None)
        for i in range(RPT // C):
            pltpu.sync_copy(s_v, den_sh.at[pl.ds(sid * RPT + i * C, C)])
        plsc.subcore_barrier()

        sems = (sem, sem2)

        def prefetch(b, buf):
            pltpu.async_copy(el_hbm.at[src_v.at[b]], el_v.at[buf], sems[buf])
            pltpu.async_copy(er_hbm.at[dst_v.at[b]], er_v.at[buf], sems[buf])

        def process(b, buf, do_prefetch):
            pltpu.make_async_copy(el_hbm.at[src_v.at[b]], el_v.at[buf],
                                  sems[buf]).wait()
            pltpu.make_async_copy(er_hbm.at[dst_v.at[b]], er_v.at[buf],
                                  sems[buf]).wait()
            if do_prefetch:
                prefetch(b + 1, 1 - buf)

            def edge(k, c2):
                z = el_v[buf, k] + er_v[buf, k]
                s_v[k] = jnp.exp(jnp.maximum(z, 0.2 * z))
                return c2
            lax.fori_loop(0, C, edge, None, unroll=2)
            pltpu.sync_copy(s_v, s_out.at[wid * CPT + b])
            pltpu.sync_copy(s_v, den_sh.at[dst_v.at[b]], add=True)

        prefetch(0, 0)

        def pair(p, carry):
            b0 = p * 2
            process(b0, 0, True)
            process(b0 + 1, 1, True)
            return carry
        lax.fori_loop(0, (CPT - 1) // 2, pair, None)
        process(CPT - 1, (CPT - 1) % 2, False)
        plsc.subcore_barrier()
        for i in range(RPT // C):
            r0 = sid * RPT + i * C
            pltpu.sync_copy(den_sh.at[pl.ds(r0, C)],
                            den_out.at[cid, pl.ds(r0, C)])

    return kern


def _make_sc_pass_b(f, h):
    nv = f // 16
    mesh = plsc.VectorSubcoreMesh(core_axis_name="c", subcore_axis_name="s")

    @functools.partial(
        pl.kernel,
        out_type=jax.ShapeDtypeStruct((2, NP, f), jnp.float32),
        mesh=mesh,
        compiler_params=pltpu.CompilerParams(use_tc_tiling_on_sc=False),
        scratch_types=[
            pltpu.VMEM((CPT, C), jnp.int32),
            pltpu.VMEM((CPT, C), jnp.int32),
            pltpu.VMEM((2, C, f), jnp.float32),
            pltpu.VMEM((C, HP), jnp.float32),
            pltpu.VMEM_SHARED((NP, f), jnp.float32),
            pltpu.SemaphoreType.DMA,
            pltpu.SemaphoreType.DMA,
        ],
    )
    def kern(feat_hbm, s_hbm, src_hbm, dst_hbm, acc_out,
             src_v, dst_v, rows_v, s_v, acc_sh, sem0, sem1):
        cid = lax.axis_index("c")
        sid = lax.axis_index("s")
        wid = cid * NSUB + sid
        sems = (sem0, sem1)
        pltpu.sync_copy(src_hbm.at[wid], src_v)
        pltpu.sync_copy(dst_hbm.at[wid], dst_v)

        def zero_body(k, carry):
            for j in range(nv):
                rows_v[0, k, pl.ds(j * 16, 16)] = jnp.zeros((16,), jnp.float32)
            return carry
        lax.fori_loop(0, C, zero_body, None)
        for i in range(RPT // C):
            pltpu.sync_copy(rows_v.at[0], acc_sh.at[pl.ds(sid * RPT + i * C, C)])
        plsc.subcore_barrier()

        pltpu.async_copy(feat_hbm.at[src_v.at[0]], rows_v.at[0], sem0)

        def process(b, buf, nxt_buf, do_prefetch):
            pltpu.make_async_copy(feat_hbm.at[src_v.at[b]],
                                  rows_v.at[buf], sems[buf]).wait()
            if do_prefetch:
                pltpu.async_copy(feat_hbm.at[src_v.at[b + 1]],
                                 rows_v.at[nxt_buf], sems[nxt_buf])
            pltpu.sync_copy(s_hbm.at[wid * CPT + b], s_v)

            def edge(k, c2):
                srow = s_v[k]
                for hh in range(h):
                    w = jnp.broadcast_to(srow[hh], (16,))
                    for j in range(2):
                        col = hh * 32 + j * 16
                        rows_v[buf, k, pl.ds(col, 16)] = (
                            rows_v[buf, k, pl.ds(col, 16)] * w)
                return c2
            lax.fori_loop(0, C, edge, None, unroll=2)
            pltpu.sync_copy(rows_v.at[buf], acc_sh.at[dst_v.at[b]], add=True)

        def pair(p, carry):
            b0 = p * 2
            process(b0, 0, 1, True)
            process(b0 + 1, 1, 0, True)
            return carry
        lax.fori_loop(0, (CPT - 1) // 2, pair, None)
        process(CPT - 1, (CPT - 1) % 2, 0, False)
        plsc.subcore_barrier()
        for i in range(RPT // C):
            r0 = sid * RPT + i * C
            pltpu.sync_copy(acc_sh.at[pl.ds(r0, C)],
                            acc_out.at[cid, pl.ds(r0, C)])

    return kern


def _make_sc_edge(f, h):
    nv = f // 16
    mesh = plsc.VectorSubcoreMesh(core_axis_name="c", subcore_axis_name="s")

    @functools.partial(
        pl.kernel,
        out_type=[
            jax.ShapeDtypeStruct((2, NP, HP), jnp.float32),
            jax.ShapeDtypeStruct((2, NP, f), jnp.float32),
        ],
        mesh=mesh,
        compiler_params=pltpu.CompilerParams(use_tc_tiling_on_sc=False),
        scratch_types=[
            pltpu.VMEM((CPT, C), jnp.int32),
            pltpu.VMEM((CPT, C), jnp.int32),
            pltpu.VMEM((2, C, f), jnp.float32),
            pltpu.VMEM((2, C, HP), jnp.float32),
            pltpu.VMEM((2, C, HP), jnp.float32),
            pltpu.VMEM((C, HP), jnp.float32),
            pltpu.VMEM_SHARED((NP, HP), jnp.float32),
            pltpu.VMEM_SHARED((NP, f), jnp.float32),
            pltpu.SemaphoreType.DMA,
            pltpu.SemaphoreType.DMA,
        ],
    )
    def kern(feat_hbm, el_hbm, er_hbm, src_hbm, dst_hbm, den_out, acc_out,
             src_v, dst_v, rows_v, el_v, er_v, s_v, den_sh, acc_sh,
             sem0, sem1):
        cid = lax.axis_index("c")
        sid = lax.axis_index("s")
        wid = cid * NSUB + sid
        sems = (sem0, sem1)
        pltpu.sync_copy(src_hbm.at[wid], src_v)
        pltpu.sync_copy(dst_hbm.at[wid], dst_v)

        def zero_body(k, carry):
            s_v[k] = jnp.zeros((HP,), jnp.float32)
            for j in range(nv):
                rows_v[0, k, pl.ds(j * 16, 16)] = jnp.zeros((16,), jnp.float32)
            return carry
        lax.fori_loop(0, C, zero_body, None)
        for i in range(RPT // C):
            r0 = sid * RPT + i * C
            pltpu.sync_copy(s_v, den_sh.at[pl.ds(r0, C)])
            pltpu.sync_copy(rows_v.at[0], acc_sh.at[pl.ds(r0, C)])
        plsc.subcore_barrier()

        def prefetch(b, buf):
            pltpu.async_copy(feat_hbm.at[src_v.at[b]], rows_v.at[buf],
                             sems[buf])
            pltpu.async_copy(el_hbm.at[src_v.at[b]], el_v.at[buf], sems[buf])
            pltpu.async_copy(er_hbm.at[dst_v.at[b]], er_v.at[buf], sems[buf])

        def wait_buf(b, buf):
            pltpu.make_async_copy(feat_hbm.at[src_v.at[b]], rows_v.at[buf],
                                  sems[buf]).wait()
            pltpu.make_async_copy(el_hbm.at[src_v.at[b]], el_v.at[buf],
                                  sems[buf]).wait()
            pltpu.make_async_copy(er_hbm.at[dst_v.at[b]], er_v.at[buf],
                                  sems[buf]).wait()

        def process(b, buf, do_prefetch):
            wait_buf(b, buf)
            if do_prefetch:
                prefetch(b + 1, 1 - buf)

            def edge(k, c2):
                z = el_v[buf, k] + er_v[buf, k]
                srow = jnp.exp(jnp.maximum(z, 0.2 * z))
                s_v[k] = srow
                for hh in range(h):
                    w = jnp.broadcast_to(srow[hh], (16,))
                    for j in range(2):
                        col = hh * 32 + j * 16
                        rows_v[buf, k, pl.ds(col, 16)] = (
                            rows_v[buf, k, pl.ds(col, 16)] * w)
                return c2
            lax.fori_loop(0, C, edge, None, unroll=2)
            pltpu.sync_copy(s_v, den_sh.at[dst_v.at[b]], add=True)
            pltpu.sync_copy(rows_v.at[buf], acc_sh.at[dst_v.at[b]], add=True)

        prefetch(0, 0)

        def pair(p, carry):
            b0 = p * 2
            process(b0, 0, True)
            process(b0 + 1, 1, True)
            return carry
        lax.fori_loop(0, (CPT - 1) // 2, pair, None)
        process(CPT - 1, (CPT - 1) % 2, False)

        plsc.subcore_barrier()
        for i in range(RPT // C):
            r0 = sid * RPT + i * C
            pltpu.sync_copy(den_sh.at[pl.ds(r0, C)],
                            den_out.at[cid, pl.ds(r0, C)])
            pltpu.sync_copy(acc_sh.at[pl.ds(r0, C)],
                            acc_out.at[cid, pl.ds(r0, C)])

    return kern


_SC_A = _make_sc_pass_a()
_SC_B128 = _make_sc_pass_b(128, 4)
_SC_EDGE = {f: _make_sc_edge(f, f // 32) for f in (96, 64)}


def kernel(node_feats, edge_index, W1, al1, ar1, g0, b0, g1, b1,
           W2, al2, ar2, g2, b2, W3, al3, ar3, Wc):
    f32 = jnp.float32
    src3d = edge_index[0].reshape(NW, CPT, C)
    dst3d = edge_index[1].reshape(NW, CPT, C)

    sel1 = jnp.asarray(_expander(4, 32, 128).T)
    sel2 = jnp.asarray(_expander(3, 32, 96).T)
    sel3 = jnp.asarray(_expander(2, 32, 64).T)
    em1 = jnp.asarray(_expander(4, 32, 128))
    em2 = jnp.asarray(_expander(3, 32, 96))
    em3 = jnp.asarray(_expander(2, 32, 64))

    ro, feat1, el1, er1 = _tc1(
        node_feats, g0.reshape(1, -1), b0.reshape(1, -1), W1,
        al1.reshape(1, -1).astype(f32), ar1.reshape(1, -1).astype(f32), sel1)
    s1, den1 = _SC_A(el1, er1, src3d, dst3d)
    acc1 = _SC_B128(feat1, s1, src3d, dst3d)

    feat2, el2, er2 = _tc_mid(
        acc1, den1, em1, g1.reshape(1, -1), b1.reshape(1, -1), W2,
        al2.reshape(1, -1).astype(f32), ar2.reshape(1, -1).astype(f32), sel2)
    den2, acc2 = _SC_EDGE[96](feat2, el2, er2, src3d, dst3d)

    feat3, el3, er3 = _tc_mid(
        acc2, den2, em2, g2.reshape(1, -1), b2.reshape(1, -1), W3,
        al3.reshape(1, -1).astype(f32), ar3.reshape(1, -1).astype(f32), sel3)
    den3, acc3 = _SC_EDGE[64](feat3, el3, er3, src3d, dst3d)

    logits = _tc_fin(acc3, den3, em3, Wc)
    return (logits, ro)
